# baseline (speedup 1.0000x reference)
import functools

import jax
import jax.numpy as jnp
import numpy as np
from jax import lax
from jax.experimental import pallas as pl
from jax.experimental.pallas import tpu as pltpu
from jax.experimental.pallas import tpu_sc as plsc

_N = 1024
_K = 64
_B = 8
_NS = 32
_EPS_TERM = float(_B * _K * 1e-31)
_BIAS = float(_N * np.log(1.0 / _B))

_NBLK = 128

_NC = 2
_NSUB = 16
_NW = _NC * _NSUB
_NPT = _N // _NW

_LUT = np.log(np.arange(16, dtype=np.float64) + _EPS_TERM).astype(np.float32)


def _tc_index_body(sample_ref, logits_ref, comp_ref, ft_ref, g_ref):
    lg = logits_ref[...]
    kio = lax.broadcasted_iota(jnp.int32, (_NBLK, _B, _K), 2)
    mx = jnp.max(lg, axis=-1, keepdims=True)
    m = jnp.min(jnp.where(lg == mx, kio, _K), axis=-1)
    cp = comp_ref[...]
    c = jnp.sum(cp * kio.astype(jnp.float32), axis=-1).astype(jnp.int32)
    t = (c - m + _K) & (_K - 1)
    nio = lax.broadcasted_iota(jnp.int32, (_NBLK, _B), 0) % _NPT
    slot = ((nio & 1) << 4) | (nio >> 1)
    tt = t + (slot << 6)
    ft_ref[...] = jnp.transpose(tt)

    smp = sample_ref[...]
    kf = (lax.broadcasted_iota(jnp.int32, (1, 1, 2 * _K), 2)
          & (_K - 1)).astype(jnp.bfloat16)
    p = smp * kf
    s_lo = jnp.sum(p[:, :, :_K], axis=-1).astype(jnp.int32)
    s_hi = jnp.sum(p[:, :, _K:], axis=-1).astype(jnp.int32)
    st_lo = jnp.transpose(s_lo)
    st_hi = jnp.transpose(s_hi)
    half = _NPT // 2
    chunks = []
    for w in range(_NBLK // _NPT):
        chunks.append(lax.slice_in_dim(st_lo, w * half, (w + 1) * half, axis=0))
        chunks.append(lax.slice_in_dim(st_hi, w * half, (w + 1) * half, axis=0))
    gt = jnp.concatenate(chunks, axis=0)
    nc = (lax.broadcasted_iota(jnp.int32, (_NBLK, _NS), 0) % _NPT) * _K
    g_ref[...] = gt + nc


def _tc_index_stage(sample_r, logits, component_probs):
    return pl.pallas_call(
        _tc_index_body,
        grid=(_N // _NBLK,),
        in_specs=[
            pl.BlockSpec((_NS, _NBLK // 2, 2 * _K), lambda i: (0, i, 0)),
            pl.BlockSpec((_NBLK, _B, _K), lambda i: (i, 0, 0)),
            pl.BlockSpec((_NBLK, _B, _K), lambda i: (i, 0, 0)),
        ],
        out_specs=[
            pl.BlockSpec((_B, _NBLK), lambda i: (0, i)),
            pl.BlockSpec((_NBLK, _NS), lambda i: (i, 0)),
        ],
        out_shape=[
            jax.ShapeDtypeStruct((_B, _N), jnp.int32),
            jax.ShapeDtypeStruct((_N, _NS), jnp.int32),
        ],
    )(sample_r, logits, component_probs)


def _sc_body(ft_hbm, g_hbm, lut_hbm, out_hbm,
             fv, gv, lut_v, tbl, acc_v, sem_f, sem_g, sem_l):
    cid = lax.axis_index("c")
    sid = lax.axis_index("s")
    wid = cid * _NSUB + sid

    n0 = wid * _NPT
    cps_f = [
        pltpu.async_copy(ft_hbm.at[b, pl.ds(n0, _NPT)],
                         fv.at[pl.ds(b * _NPT, _NPT)], sem_f)
        for b in range(_B)
    ]
    cp_g = pltpu.async_copy(g_hbm.at[pl.ds(wid * _NPT * _NS, _NPT * _NS)],
                            gv, sem_g)
    cp_l = pltpu.async_copy(lut_hbm, lut_v, sem_l)

    zeros = jnp.zeros((16,), jnp.float32)
    for i in range(_NPT * _K // 16):
        tbl[pl.ds(i * 16, 16)] = zeros

    for cp in cps_f:
        cp.wait()
    ones = jnp.ones((16,), jnp.float32)
    for b in range(_B):
        for h in range(_NPT // 16):
            idx = fv[pl.ds(b * _NPT + h * 16, 16)]
            plsc.addupdate_scatter(tbl, [idx], ones)

    cp_l.wait()
    cp_g.wait()
    acc0 = jnp.zeros((16,), jnp.float32)
    acc1 = jnp.zeros((16,), jnp.float32)
    for n in range(_NPT):
        g0 = gv[pl.ds(n * _NS, 16)]
        g1 = gv[pl.ds(n * _NS + 16, 16)]
        c0 = plsc.load_gather(tbl, [g0]).astype(jnp.int32)
        c1 = plsc.load_gather(tbl, [g1]).astype(jnp.int32)
        acc0 = acc0 + plsc.load_gather(lut_v, [c0])
        acc1 = acc1 + plsc.load_gather(lut_v, [c1])
    acc_v[pl.ds(0, 16)] = acc0
    acc_v[pl.ds(16, 16)] = acc1
    pltpu.sync_copy(acc_v, out_hbm.at[wid])


def _sc_stage(ft2d, g_flat, lut):
    fn = pl.kernel(
        _sc_body,
        out_type=jax.ShapeDtypeStruct((_NW, _NS), jnp.float32),
        mesh=plsc.VectorSubcoreMesh(core_axis_name="c", subcore_axis_name="s",
                                    num_cores=_NC, num_subcores=_NSUB),
        scratch_types=[
            pltpu.VMEM((_B * _NPT,), jnp.int32),
            pltpu.VMEM((_NPT * _NS,), jnp.int32),
            pltpu.VMEM((16,), jnp.float32),
            pltpu.VMEM((_NPT * _K,), jnp.float32),
            pltpu.VMEM((_NS,), jnp.float32),
            pltpu.SemaphoreType.DMA,
            pltpu.SemaphoreType.DMA,
            pltpu.SemaphoreType.DMA,
        ],
        compiler_params=pltpu.CompilerParams(needs_layout_passes=False),
    )
    return fn(ft2d, g_flat, lut)


@jax.jit
def kernel(sample, logits, component_probs):
    sample_r = sample.astype(jnp.bfloat16).reshape(_NS, _N // 2, 2 * _K)
    ft2d, g2d = _tc_index_stage(sample_r, logits, component_probs)
    lut = jnp.asarray(_LUT)
    partial = _sc_stage(ft2d, g2d.reshape(_N * _NS), lut)
    return jnp.sum(partial, axis=0) + _BIAS


# speedup vs baseline: 1.1559x; 1.1559x over previous
import functools

import jax
import jax.numpy as jnp
import numpy as np
from jax import lax
from jax.experimental import pallas as pl
from jax.experimental.pallas import tpu as pltpu
from jax.experimental.pallas import tpu_sc as plsc

_N = 1024
_K = 64
_B = 8
_NS = 32
_EPS_TERM = float(_B * _K * 1e-31)
_BIAS = float(_N * np.log(1.0 / _B))

_NBLK = 128

_NC = 2
_NSUB = 16
_NW = _NC * _NSUB
_NPT = _N // _NW

_LUT = np.log(np.arange(16, dtype=np.float64) + _EPS_TERM).astype(np.float32)


def _tc_index_body(sample_ref, logits_ref, comp_ref, ft_ref, g_ref):
    lg = logits_ref[...]
    kio = lax.broadcasted_iota(jnp.int32, (_NBLK, _B, _K), 2)
    mx = jnp.max(lg, axis=-1, keepdims=True)
    m = jnp.min(jnp.where(lg == mx, kio, _K), axis=-1)
    cp = comp_ref[...]
    c = jnp.sum(cp * kio.astype(jnp.float32), axis=-1).astype(jnp.int32)
    t = (c - m + _K) & (_K - 1)
    nio = lax.broadcasted_iota(jnp.int32, (_NBLK, _B), 0) % _NPT
    tt = t + (nio << 6)
    ft_ref[...] = jnp.transpose(tt)

    smp = sample_ref[...]
    kio_s = lax.broadcasted_iota(jnp.int32, (_NS, _NBLK, _K), 2)
    s = jnp.sum(smp * kio_s.astype(jnp.float32),
                axis=-1).astype(jnp.int32)
    nc = (lax.broadcasted_iota(jnp.int32, (_NBLK, _NS), 0) % _NPT) * _K
    g_ref[:, : _NS] = jnp.transpose(s) + nc


def _tc_index_stage(sample, logits, component_probs):
    return pl.pallas_call(
        _tc_index_body,
        grid=(_N // _NBLK,),
        in_specs=[
            pl.BlockSpec((_NS, _NBLK, _K), lambda i: (0, i, 0)),
            pl.BlockSpec((_NBLK, _B, _K), lambda i: (i, 0, 0)),
            pl.BlockSpec((_NBLK, _B, _K), lambda i: (i, 0, 0)),
        ],
        out_specs=[
            pl.BlockSpec((_B, _NBLK), lambda i: (0, i)),
            pl.BlockSpec((_NBLK, 128), lambda i: (i, 0)),
        ],
        out_shape=[
            jax.ShapeDtypeStruct((_B, _N), jnp.int32),
            jax.ShapeDtypeStruct((_N, 128), jnp.int32),
        ],
    )(sample, logits, component_probs)


def _sc_body(ft_hbm, g_hbm, lut_hbm, out_hbm,
             fv, gv, lut_v, tbl, acc_v, sem_f, sem_g, sem_l):
    cid = lax.axis_index("c")
    sid = lax.axis_index("s")
    wid = cid * _NSUB + sid

    n0 = wid * _NPT
    cps_f = [
        pltpu.async_copy(ft_hbm.at[b, pl.ds(n0, _NPT)],
                         fv.at[pl.ds(b * _NPT, _NPT)], sem_f)
        for b in range(_B)
    ]
    cp_g = pltpu.async_copy(g_hbm.at[pl.ds(wid * _NPT * 128, _NPT * 128)],
                            gv, sem_g)
    cp_l = pltpu.async_copy(lut_hbm, lut_v, sem_l)

    zeros = jnp.zeros((16,), jnp.float32)
    for i in range(_NPT * _K // 16):
        tbl[pl.ds(i * 16, 16)] = zeros

    for cp in cps_f:
        cp.wait()
    ones = jnp.ones((16,), jnp.float32)
    for b in range(_B):
        for h in range(_NPT // 16):
            idx = fv[pl.ds(b * _NPT + h * 16, 16)]
            plsc.addupdate_scatter(tbl, [idx], ones)

    cp_l.wait()
    cp_g.wait()
    acc0 = jnp.zeros((16,), jnp.float32)
    acc1 = jnp.zeros((16,), jnp.float32)
    for n in range(_NPT):
        g0 = gv[pl.ds(n * 128, 16)]
        g1 = gv[pl.ds(n * 128 + 16, 16)]
        c0 = plsc.load_gather(tbl, [g0]).astype(jnp.int32)
        c1 = plsc.load_gather(tbl, [g1]).astype(jnp.int32)
        acc0 = acc0 + plsc.load_gather(lut_v, [c0])
        acc1 = acc1 + plsc.load_gather(lut_v, [c1])
    acc_v[pl.ds(0, 16)] = acc0
    acc_v[pl.ds(16, 16)] = acc1
    pltpu.sync_copy(acc_v, out_hbm.at[wid])


def _sc_stage(ft2d, g_flat, lut):
    fn = pl.kernel(
        _sc_body,
        out_type=jax.ShapeDtypeStruct((_NW, _NS), jnp.float32),
        mesh=plsc.VectorSubcoreMesh(core_axis_name="c", subcore_axis_name="s",
                                    num_cores=_NC, num_subcores=_NSUB),
        scratch_types=[
            pltpu.VMEM((_B * _NPT,), jnp.int32),
            pltpu.VMEM((_NPT * 128,), jnp.int32),
            pltpu.VMEM((16,), jnp.float32),
            pltpu.VMEM((_NPT * _K,), jnp.float32),
            pltpu.VMEM((_NS,), jnp.float32),
            pltpu.SemaphoreType.DMA,
            pltpu.SemaphoreType.DMA,
            pltpu.SemaphoreType.DMA,
        ],
        compiler_params=pltpu.CompilerParams(needs_layout_passes=False),
    )
    return fn(ft2d, g_flat, lut)


@jax.jit
def kernel(sample, logits, component_probs):
    ft2d, g2d = _tc_index_stage(sample, logits, component_probs)
    lut = jnp.asarray(_LUT)
    partial = _sc_stage(ft2d, g2d.reshape(_N * 128), lut)
    return jnp.sum(partial, axis=0) + _BIAS
